# baseline (speedup 1.0000x reference)
import functools

import jax
import jax.numpy as jnp
from jax import lax
from jax.experimental import pallas as pl
from jax.experimental.pallas import tpu as pltpu
from jax.experimental.pallas import tpu_sc as plsc

N_NODES = 10000
N_EDGES = 320000
D = 128
D_EDGE = 16
NUM_GRAPHS = 512
H = 4
TEMPERATURE = 0.1
EPS = 1e-12

NC = 2
NS = 16
NW = NC * NS

K1 = 128
NCH = N_EDGES // K1
CPW = NCH // NW
CPW_X = NCH - CPW * NW

EPC = N_EDGES // NC
NBLK_S = EPC // K1
NH = 2
NHN = N_NODES // NH
FG = 16
ACC_W = NHN * FG
ZW = 8192


def _sc_gather_impl(src_hbm, x_hbm, eaw_hbm, mr_hbm, sidx_c, rows, eawb, sem):
    wid = lax.axis_index("c") * NS + lax.axis_index("s")
    start = wid * CPW + jnp.minimum(wid, CPW_X)
    nch = CPW + jnp.where(wid < CPW_X, 1, 0)

    def _chunk(j, _):
        e0 = (start + j) * K1
        pltpu.sync_copy(src_hbm.at[pl.ds(e0, K1)], sidx_c)
        g = pltpu.async_copy(x_hbm.at[sidx_c], rows, sem)
        pltpu.sync_copy(eaw_hbm.at[pl.ds(e0, K1)], eawb)
        g.wait()

        def _addrow(r, _2):
            for q in range(D // 16):
                sl = pl.ds(q * 16, 16)
                rows[r, sl] = rows[r, sl] + eawb[r, sl]
            return _2

        lax.fori_loop(0, K1, _addrow, None)
        pltpu.sync_copy(rows, mr_hbm.at[pl.ds(e0, K1)])
        return _

    lax.fori_loop(0, nch, _chunk, None)


@functools.cache
def _sc_gather():
    mesh = plsc.VectorSubcoreMesh(core_axis_name="c", subcore_axis_name="s",
                                  num_cores=NC, num_subcores=NS)
    return pl.kernel(
        _sc_gather_impl,
        out_type=jax.ShapeDtypeStruct((N_EDGES, D), jnp.float32),
        mesh=mesh,
        scratch_types=[
            pltpu.VMEM((K1,), jnp.int32),
            pltpu.VMEM((K1, D), jnp.float32),
            pltpu.VMEM((K1, D), jnp.float32),
            pltpu.SemaphoreType.DMA,
        ],
    )


def _sc_scatter_impl(mr_hbm, dst3_hbm, zr_hbm, aggq_hbm,
                     didx_a, didx_b, stage_a, stage_b, acc2, sem_a, sem_b):
    c = lax.axis_index("c")
    s = lax.axis_index("s")
    nh = s // 8
    fg = s % 8
    lo16 = nh * NHN * FG
    f0 = fg * FG
    iota16 = lax.broadcasted_iota(jnp.int32, (16,), 0)

    for k in range(ACC_W // ZW):
        pltpu.sync_copy(zr_hbm, acc2.at[pl.ds(k * ZW, ZW)])
    rem = ACC_W - (ACC_W // ZW) * ZW
    pltpu.sync_copy(zr_hbm.at[pl.ds(0, rem)],
                    acc2.at[pl.ds(ACC_W - rem, rem)])

    b_base = c * NBLK_S

    def _start(b, didx, stage, sem):
        pltpu.async_copy(dst3_hbm.at[c, b], didx, sem)
        pltpu.async_copy(mr_hbm.at[pl.ds((b_base + b) * K1, K1)], stage, sem)

    def _drain(didx, stage, sem):
        pltpu.make_async_copy(dst3_hbm.at[0, 0], didx, sem).wait()
        pltpu.make_async_copy(mr_hbm.at[pl.ds(0, K1)], stage, sem).wait()

    def _process(didx, stage):
        for g in range(K1 // 16):
            dv = didx[pl.ds(g * 16, 16)]
            for p in range(16):
                d = dv[p]
                idx = d * FG - lo16 + iota16
                msk = (idx >= 0) & (idx < ACC_W)
                vals = stage[g * 16 + p, pl.ds(f0, FG)]
                plsc.addupdate_scatter(acc2, [idx], vals, mask=msk)

    _start(0, didx_a, stage_a, sem_a)

    def _body(t, _):
        _drain(didx_a, stage_a, sem_a)
        _start(2 * t + 1, didx_b, stage_b, sem_b)
        _process(didx_a, stage_a)
        _drain(didx_b, stage_b, sem_b)

        @pl.when(t < NBLK_S // 2 - 1)
        def _next():
            _start(2 * t + 2, didx_a, stage_a, sem_a)

        _process(didx_b, stage_b)
        return _

    lax.fori_loop(0, NBLK_S // 2, _body, None)
    pltpu.sync_copy(acc2, aggq_hbm.at[c, s])


@functools.cache
def _sc_scatter():
    mesh = plsc.VectorSubcoreMesh(core_axis_name="c", subcore_axis_name="s",
                                  num_cores=NC, num_subcores=NS)
    return pl.kernel(
        _sc_scatter_impl,
        out_type=jax.ShapeDtypeStruct((NC, NS, ACC_W), jnp.float32),
        mesh=mesh,
        compiler_params=pltpu.CompilerParams(needs_layout_passes=False),
        scratch_types=[
            pltpu.VMEM((K1,), jnp.int32),
            pltpu.VMEM((K1,), jnp.int32),
            pltpu.VMEM((K1, D), jnp.float32),
            pltpu.VMEM((K1, D), jnp.float32),
            pltpu.VMEM((ACC_W,), jnp.float32),
            pltpu.SemaphoreType.DMA,
            pltpu.SemaphoreType.DMA,
        ],
    )


EAB = 2000


def _tc_eaw_body(ea_ref, We_ref, out_ref):
    out_ref[...] = jnp.dot(ea_ref[...], We_ref[...],
                           preferred_element_type=jnp.float32)


_tc_eaw = pl.pallas_call(
    _tc_eaw_body,
    grid=(N_EDGES // EAB,),
    in_specs=[
        pl.BlockSpec((EAB, D_EDGE), lambda i: (i, 0)),
        pl.BlockSpec((D_EDGE, D), lambda i: (0, 0)),
    ],
    out_specs=pl.BlockSpec((EAB, D), lambda i: (i, 0)),
    out_shape=jax.ShapeDtypeStruct((N_EDGES, D), jnp.float32),
)


NB = 1000
NBLK = N_NODES // NB


def _tc_body(gid_ref, x_ref, ax_ref, b_ref, Wg_ref, bg_ref,
             W1_ref, W2_ref, out_ref, acc_h, acc_m, acc_c):
    i = pl.program_id(0)

    @pl.when(i == 0)
    def _init():
        acc_h[...] = jnp.zeros_like(acc_h)
        acc_m[...] = jnp.zeros_like(acc_m)
        acc_c[...] = jnp.zeros_like(acc_c)

    a = x_ref[...] + ax_ref[0] + ax_ref[1]
    h = jnp.maximum(
        jnp.dot(a, Wg_ref[...], preferred_element_type=jnp.float32)
        + bg_ref[...], 0.0)
    r = jnp.maximum(jnp.dot(h, W1_ref[...], preferred_element_type=jnp.float32),
                    0.0)
    m = h + r

    bt = b_ref[0]
    gids = lax.broadcasted_iota(jnp.int32, (NUM_GRAPHS, NB), 0)
    onehot = (bt == gids).astype(jnp.float32)
    acc_h[...] += jnp.dot(onehot, h, preferred_element_type=jnp.float32)
    acc_m[...] += jnp.dot(onehot, m, preferred_element_type=jnp.float32)
    acc_c[...] += jnp.dot(onehot, jnp.ones((NB, D), jnp.float32),
                          preferred_element_type=jnp.float32)

    @pl.when(i == NBLK - 1)
    def _finish():
        cnt = jnp.maximum(acc_c[...], 1.0)
        g_h = acc_h[...] / cnt
        pm = acc_m[...] / cnt

        def _norm(v):
            n = jnp.sqrt(jnp.sum(v * v, axis=1, keepdims=True))
            return v / jnp.maximum(n, EPS)

        out1 = _norm(pm)
        z = jnp.dot(g_h, W2_ref[...], preferred_element_type=jnp.float32)
        o2 = (z[:, 0:D] + z[:, D:2 * D] + z[:, 2 * D:3 * D]
              + z[:, 3 * D:4 * D]) * (1.0 / H)
        out2 = _norm(o2)
        sim = jnp.sum(out1 * out2, axis=1, keepdims=True)
        t = sim / TEMPERATURE
        ridx = lax.broadcasted_iota(jnp.int32, (NUM_GRAPHS, 1), 0)
        masked = jnp.where(ridx == 3, -jnp.inf, t)
        mx = jnp.max(masked)
        lse = jnp.log(jnp.sum(jnp.exp(masked - mx))) + mx
        partner = (gid_ref[0] + NUM_GRAPHS // 2) % NUM_GRAPHS
        tp = jnp.sum(jnp.where(ridx == partner, t, 0.0))
        out_ref[...] = (lse - tp).reshape(1, 1)


_tc_dense = pl.pallas_call(
    _tc_body,
    grid=(NBLK,),
    in_specs=[
        pl.BlockSpec(memory_space=pltpu.SMEM),
        pl.BlockSpec((NB, D), lambda i: (i, 0)),
        pl.BlockSpec((NC, NB, D), lambda i: (0, i, 0)),
        pl.BlockSpec((1, 1, NB), lambda i: (i, 0, 0)),
        pl.BlockSpec((D, D), lambda i: (0, 0)),
        pl.BlockSpec((1, D), lambda i: (0, 0)),
        pl.BlockSpec((D, D), lambda i: (0, 0)),
        pl.BlockSpec((D, H * D), lambda i: (0, 0)),
    ],
    out_specs=pl.BlockSpec((1, 1), lambda i: (0, 0)),
    out_shape=jax.ShapeDtypeStruct((1, 1), jnp.float32),
    scratch_shapes=[
        pltpu.VMEM((NUM_GRAPHS, D), jnp.float32),
        pltpu.VMEM((NUM_GRAPHS, D), jnp.float32),
        pltpu.VMEM((NUM_GRAPHS, D), jnp.float32),
    ],
)


def kernel(gid, x, edge_index, edge_attr, batch, W_e, W_g, b_g, W1, W2):
    src = edge_index[0]
    dst3 = edge_index[1].reshape(NC, NBLK_S, K1)
    zr = jnp.zeros((ZW,), jnp.float32)

    eaw = _tc_eaw(edge_attr, W_e)
    mr = _sc_gather()(src, x, eaw)
    aggq = _sc_scatter()(mr, dst3, zr)
    aggx = aggq.reshape(NC, NH, 8, NHN, FG).transpose(0, 1, 3, 2, 4)
    aggx = aggx.reshape(NC, N_NODES, D)

    gid_arr = jnp.asarray(gid, jnp.int32).reshape(1)
    batch3 = batch.reshape(NBLK, 1, NB)
    loss = _tc_dense(gid_arr, x, aggx, batch3, W_g,
                     b_g.reshape(1, D), W1, W2)
    return loss.reshape(())


# speedup vs baseline: 1.0000x; 1.0000x over previous
import functools

import jax
import jax.numpy as jnp
from jax import lax
from jax.experimental import pallas as pl
from jax.experimental.pallas import tpu as pltpu
from jax.experimental.pallas import tpu_sc as plsc

N_NODES = 10000
N_EDGES = 320000
D = 128
D_EDGE = 16
NUM_GRAPHS = 512
H = 4
TEMPERATURE = 0.1
EPS = 1e-12

NC = 2
NS = 16
NW = NC * NS

K1 = 128
NCH = N_EDGES // K1
CPW = NCH // NW
CPW_X = NCH - CPW * NW

EPC = N_EDGES // NC
NBLK_S = EPC // K1
NH = 2
NHN = N_NODES // NH
FG = 16
ACC_W = NHN * FG
ZW = 8192


def _sc_gather_impl(src_hbm, x_hbm, eaw_hbm, mr_hbm, sidx_c, rows, eawb, sem):
    wid = lax.axis_index("c") * NS + lax.axis_index("s")
    start = wid * CPW + jnp.minimum(wid, CPW_X)
    nch = CPW + jnp.where(wid < CPW_X, 1, 0)

    def _chunk(j, _):
        e0 = (start + j) * K1
        pltpu.sync_copy(src_hbm.at[pl.ds(e0, K1)], sidx_c)
        g = pltpu.async_copy(x_hbm.at[sidx_c], rows, sem)
        pltpu.sync_copy(eaw_hbm.at[pl.ds(e0, K1)], eawb)
        g.wait()

        def _addrow(r, _2):
            for q in range(D // 16):
                sl = pl.ds(q * 16, 16)
                rows[r, sl] = rows[r, sl] + eawb[r, sl]
            return _2

        lax.fori_loop(0, K1, _addrow, None)
        pltpu.sync_copy(rows, mr_hbm.at[pl.ds(e0, K1)])
        return _

    lax.fori_loop(0, nch, _chunk, None)


@functools.cache
def _sc_gather():
    mesh = plsc.VectorSubcoreMesh(core_axis_name="c", subcore_axis_name="s",
                                  num_cores=NC, num_subcores=NS)
    return pl.kernel(
        _sc_gather_impl,
        out_type=jax.ShapeDtypeStruct((N_EDGES, D), jnp.float32),
        mesh=mesh,
        scratch_types=[
            pltpu.VMEM((K1,), jnp.int32),
            pltpu.VMEM((K1, D), jnp.float32),
            pltpu.VMEM((K1, D), jnp.float32),
            pltpu.SemaphoreType.DMA,
        ],
    )


def _sc_scatter_impl(mr_hbm, dst3_hbm, zr_hbm, aggq_hbm,
                     didx_a, didx_b, stage_a, stage_b, acc2, sem_a, sem_b):
    c = lax.axis_index("c")
    s = lax.axis_index("s")
    nh = s // 8
    fg = s % 8
    lo16 = nh * NHN * FG
    f0 = fg * FG
    iota16 = lax.broadcasted_iota(jnp.int32, (16,), 0)
    ioff = iota16 - lo16
    dnums = lax.GatherDimensionNumbers(offset_dims=(),
                                       collapsed_slice_dims=(0,),
                                       start_index_map=(0,))
    pats = [jnp.full((16, 1), p, jnp.int32) for p in range(16)]

    for k in range(ACC_W // ZW):
        pltpu.sync_copy(zr_hbm, acc2.at[pl.ds(k * ZW, ZW)])
    rem = ACC_W - (ACC_W // ZW) * ZW
    pltpu.sync_copy(zr_hbm.at[pl.ds(0, rem)],
                    acc2.at[pl.ds(ACC_W - rem, rem)])

    b_base = c * NBLK_S

    def _start(b, didx, stage, sem):
        pltpu.async_copy(dst3_hbm.at[c, b], didx, sem)
        pltpu.async_copy(mr_hbm.at[pl.ds((b_base + b) * K1, K1)], stage, sem)

    def _drain(didx, stage, sem):
        pltpu.make_async_copy(dst3_hbm.at[0, 0], didx, sem).wait()
        pltpu.make_async_copy(mr_hbm.at[pl.ds(0, K1)], stage, sem).wait()

    def _process(didx, stage):
        for g in range(K1 // 16):
            dv = didx[pl.ds(g * 16, 16)]
            for p in range(16):
                db = lax.gather(dv, pats[p], dnums, slice_sizes=(1,),
                                mode=lax.GatherScatterMode.PROMISE_IN_BOUNDS)
                idx = db * FG + ioff
                msk = (idx >= 0) & (idx < ACC_W)
                vals = stage[g * 16 + p, pl.ds(f0, FG)]
                plsc.addupdate_scatter(acc2, [idx], vals, mask=msk)

    _start(0, didx_a, stage_a, sem_a)

    def _body(t, _):
        _drain(didx_a, stage_a, sem_a)
        _start(2 * t + 1, didx_b, stage_b, sem_b)
        _process(didx_a, stage_a)
        _drain(didx_b, stage_b, sem_b)

        @pl.when(t < NBLK_S // 2 - 1)
        def _next():
            _start(2 * t + 2, didx_a, stage_a, sem_a)

        _process(didx_b, stage_b)
        return _

    lax.fori_loop(0, NBLK_S // 2, _body, None)
    pltpu.sync_copy(acc2, aggq_hbm.at[c, s])


@functools.cache
def _sc_scatter():
    mesh = plsc.VectorSubcoreMesh(core_axis_name="c", subcore_axis_name="s",
                                  num_cores=NC, num_subcores=NS)
    return pl.kernel(
        _sc_scatter_impl,
        out_type=jax.ShapeDtypeStruct((NC, NS, ACC_W), jnp.float32),
        mesh=mesh,
        compiler_params=pltpu.CompilerParams(needs_layout_passes=False),
        scratch_types=[
            pltpu.VMEM((K1,), jnp.int32),
            pltpu.VMEM((K1,), jnp.int32),
            pltpu.VMEM((K1, D), jnp.float32),
            pltpu.VMEM((K1, D), jnp.float32),
            pltpu.VMEM((ACC_W,), jnp.float32),
            pltpu.SemaphoreType.DMA,
            pltpu.SemaphoreType.DMA,
        ],
    )


EAB = 2000


def _tc_eaw_body(ea_ref, We_ref, out_ref):
    out_ref[...] = jnp.dot(ea_ref[...], We_ref[...],
                           preferred_element_type=jnp.float32)


_tc_eaw = pl.pallas_call(
    _tc_eaw_body,
    grid=(N_EDGES // EAB,),
    in_specs=[
        pl.BlockSpec((EAB, D_EDGE), lambda i: (i, 0)),
        pl.BlockSpec((D_EDGE, D), lambda i: (0, 0)),
    ],
    out_specs=pl.BlockSpec((EAB, D), lambda i: (i, 0)),
    out_shape=jax.ShapeDtypeStruct((N_EDGES, D), jnp.float32),
)


NB = 1000
NBLK = N_NODES // NB


def _tc_body(gid_ref, x_ref, ax_ref, b_ref, Wg_ref, bg_ref,
             W1_ref, W2_ref, out_ref, acc_h, acc_m, acc_c):
    i = pl.program_id(0)

    @pl.when(i == 0)
    def _init():
        acc_h[...] = jnp.zeros_like(acc_h)
        acc_m[...] = jnp.zeros_like(acc_m)
        acc_c[...] = jnp.zeros_like(acc_c)

    a = x_ref[...] + ax_ref[0] + ax_ref[1]
    h = jnp.maximum(
        jnp.dot(a, Wg_ref[...], preferred_element_type=jnp.float32)
        + bg_ref[...], 0.0)
    r = jnp.maximum(jnp.dot(h, W1_ref[...], preferred_element_type=jnp.float32),
                    0.0)
    m = h + r

    bt = b_ref[0]
    gids = lax.broadcasted_iota(jnp.int32, (NUM_GRAPHS, NB), 0)
    onehot = (bt == gids).astype(jnp.float32)
    acc_h[...] += jnp.dot(onehot, h, preferred_element_type=jnp.float32)
    acc_m[...] += jnp.dot(onehot, m, preferred_element_type=jnp.float32)
    acc_c[...] += jnp.dot(onehot, jnp.ones((NB, D), jnp.float32),
                          preferred_element_type=jnp.float32)

    @pl.when(i == NBLK - 1)
    def _finish():
        cnt = jnp.maximum(acc_c[...], 1.0)
        g_h = acc_h[...] / cnt
        pm = acc_m[...] / cnt

        def _norm(v):
            n = jnp.sqrt(jnp.sum(v * v, axis=1, keepdims=True))
            return v / jnp.maximum(n, EPS)

        out1 = _norm(pm)
        z = jnp.dot(g_h, W2_ref[...], preferred_element_type=jnp.float32)
        o2 = (z[:, 0:D] + z[:, D:2 * D] + z[:, 2 * D:3 * D]
              + z[:, 3 * D:4 * D]) * (1.0 / H)
        out2 = _norm(o2)
        sim = jnp.sum(out1 * out2, axis=1, keepdims=True)
        t = sim / TEMPERATURE
        ridx = lax.broadcasted_iota(jnp.int32, (NUM_GRAPHS, 1), 0)
        masked = jnp.where(ridx == 3, -jnp.inf, t)
        mx = jnp.max(masked)
        lse = jnp.log(jnp.sum(jnp.exp(masked - mx))) + mx
        partner = (gid_ref[0] + NUM_GRAPHS // 2) % NUM_GRAPHS
        tp = jnp.sum(jnp.where(ridx == partner, t, 0.0))
        out_ref[...] = (lse - tp).reshape(1, 1)


_tc_dense = pl.pallas_call(
    _tc_body,
    grid=(NBLK,),
    in_specs=[
        pl.BlockSpec(memory_space=pltpu.SMEM),
        pl.BlockSpec((NB, D), lambda i: (i, 0)),
        pl.BlockSpec((NC, NB, D), lambda i: (0, i, 0)),
        pl.BlockSpec((1, 1, NB), lambda i: (i, 0, 0)),
        pl.BlockSpec((D, D), lambda i: (0, 0)),
        pl.BlockSpec((1, D), lambda i: (0, 0)),
        pl.BlockSpec((D, D), lambda i: (0, 0)),
        pl.BlockSpec((D, H * D), lambda i: (0, 0)),
    ],
    out_specs=pl.BlockSpec((1, 1), lambda i: (0, 0)),
    out_shape=jax.ShapeDtypeStruct((1, 1), jnp.float32),
    scratch_shapes=[
        pltpu.VMEM((NUM_GRAPHS, D), jnp.float32),
        pltpu.VMEM((NUM_GRAPHS, D), jnp.float32),
        pltpu.VMEM((NUM_GRAPHS, D), jnp.float32),
    ],
)


def kernel(gid, x, edge_index, edge_attr, batch, W_e, W_g, b_g, W1, W2):
    src = edge_index[0]
    dst3 = edge_index[1].reshape(NC, NBLK_S, K1)
    zr = jnp.zeros((ZW,), jnp.float32)

    eaw = _tc_eaw(edge_attr, W_e)
    mr = _sc_gather()(src, x, eaw)
    aggq = _sc_scatter()(mr, dst3, zr)
    aggx = aggq.reshape(NC, NH, 8, NHN, FG).transpose(0, 1, 3, 2, 4)
    aggx = aggx.reshape(NC, N_NODES, D)

    gid_arr = jnp.asarray(gid, jnp.int32).reshape(1)
    batch3 = batch.reshape(NBLK, 1, NB)
    loss = _tc_dense(gid_arr, x, aggx, batch3, W_g,
                     b_g.reshape(1, D), W1, W2)
    return loss.reshape(())
